# Initial kernel scaffold; baseline (speedup 1.0000x reference)
import jax
import jax.numpy as jnp
from jax.experimental import pallas as pl


def kernel(x, table):
    raise NotImplementedError("write your pallas kernel here")


# speedup vs baseline: 2.4885x; 2.4885x over previous
import functools

import jax
import jax.numpy as jnp
from jax import lax
from jax.experimental import pallas as pl
from jax.experimental.pallas import tpu as pltpu
from jax.experimental.pallas import tpu_sc as plsc

NC, NS = 2, 16
NW = NC * NS
CHUNK = 128


def kernel(x, table):
    batch, seq = x.shape
    _, embed_dim = table.shape
    num_indices = batch * seq
    per_worker = num_indices // NW
    n_chunks = per_worker // CHUNK
    idx3d = x.reshape(NW, n_chunks, CHUNK)

    mesh = plsc.VectorSubcoreMesh(core_axis_name="c", subcore_axis_name="s")

    @functools.partial(
        pl.kernel,
        mesh=mesh,
        out_type=jax.ShapeDtypeStruct((num_indices, embed_dim), table.dtype),
        scratch_types=[
            pltpu.VMEM((n_chunks, CHUNK), jnp.int32),
            pltpu.VMEM((CHUNK, embed_dim), jnp.float32),
            pltpu.SemaphoreType.DMA,
        ],
        compiler_params=pltpu.CompilerParams(use_tc_tiling_on_sc=False),
    )
    def sc_gather(table_hbm, idx_hbm, out_hbm, idx_v, rows_v, sem):
        wid = lax.axis_index("s") * NC + lax.axis_index("c")
        base = wid * per_worker
        pltpu.sync_copy(idx_hbm.at[wid], idx_v)

        @pl.loop(0, n_chunks)
        def _(j):
            pltpu.async_copy(table_hbm.at[idx_v.at[j]], rows_v, sem).wait()
            pltpu.sync_copy(rows_v, out_hbm.at[pl.ds(base + j * CHUNK, CHUNK)])

    out = sc_gather(table, idx3d)
    return out.reshape(batch, seq, embed_dim)


# speedup vs baseline: 2.7304x; 1.0972x over previous
import functools

import jax
import jax.numpy as jnp
from jax import lax
from jax.experimental import pallas as pl
from jax.experimental.pallas import tpu as pltpu
from jax.experimental.pallas import tpu_sc as plsc

NC, NS = 2, 16
NW = NC * NS
CHUNK = 128
K = 5


def kernel(x, table):
    batch, seq = x.shape
    _, embed_dim = table.shape
    num_indices = batch * seq
    per_worker = num_indices // NW
    n_chunks = per_worker // CHUNK
    n_batches = n_chunks // K
    rows_per_batch = K * CHUNK
    idx3d = x.reshape(NW, n_chunks, CHUNK)

    mesh = plsc.VectorSubcoreMesh(core_axis_name="c", subcore_axis_name="s")

    @functools.partial(
        pl.kernel,
        mesh=mesh,
        out_type=jax.ShapeDtypeStruct((num_indices, embed_dim), table.dtype),
        scratch_types=[
            pltpu.VMEM((n_chunks, CHUNK), jnp.int32),
            pltpu.VMEM((2, rows_per_batch, embed_dim), jnp.float32),
            pltpu.SemaphoreType.DMA,
            pltpu.SemaphoreType.DMA,
            pltpu.SemaphoreType.DMA,
            pltpu.SemaphoreType.DMA,
        ],
        compiler_params=pltpu.CompilerParams(use_tc_tiling_on_sc=False),
    )
    def sc_gather(table_hbm, idx_hbm, out_hbm, idx_v, rows_v, g0, g1, o0, o1):
        wid = lax.axis_index("s") * NC + lax.axis_index("c")
        base = wid * per_worker
        gsem = (g0, g1)
        osem = (o0, o1)
        pltpu.sync_copy(idx_hbm.at[wid], idx_v)

        def gather_cp(buf, t, i):
            return pltpu.make_async_copy(
                table_hbm.at[idx_v.at[t * K + i]],
                rows_v.at[buf].at[pl.ds(i * CHUNK, CHUNK)],
                gsem[buf],
            )

        def out_cp(buf, t):
            return pltpu.make_async_copy(
                rows_v.at[buf],
                out_hbm.at[pl.ds(base + t * rows_per_batch, rows_per_batch)],
                osem[buf],
            )

        def fire(buf, t):
            for i in range(K):
                gather_cp(buf, t, i).start()

        def drain(buf, t):
            for i in range(K):
                gather_cp(buf, t, i).wait()

        fire(0, 0)
        fire(1, 1)

        @pl.loop(0, n_batches // 2 - 1)
        def _(h):
            t0 = 2 * h
            drain(0, t0)
            out_cp(0, t0).start()
            drain(1, t0 + 1)
            out_cp(1, t0 + 1).start()
            out_cp(0, t0).wait()
            fire(0, t0 + 2)
            out_cp(1, t0 + 1).wait()
            fire(1, t0 + 3)

        tl = n_batches - 2
        drain(0, tl)
        out_cp(0, tl).start()
        drain(1, tl + 1)
        out_cp(1, tl + 1).start()
        out_cp(0, tl).wait()
        out_cp(1, tl + 1).wait()

    out = sc_gather(table, idx3d)
    return out.reshape(batch, seq, embed_dim)


# speedup vs baseline: 2.7450x; 1.0053x over previous
import functools

import jax
import jax.numpy as jnp
from jax import lax
from jax.experimental import pallas as pl
from jax.experimental.pallas import tpu as pltpu
from jax.experimental.pallas import tpu_sc as plsc

NC, NS = 2, 16
NW = NC * NS
W = 1600


def kernel(x, table):
    batch, seq = x.shape
    vocab, embed_dim = table.shape
    num_indices = batch * seq
    per_worker = num_indices // NW
    n_batches = per_worker // W

    idx_flat = x.reshape(num_indices)

    mesh = plsc.VectorSubcoreMesh(core_axis_name="c", subcore_axis_name="s")

    @functools.partial(
        pl.kernel,
        mesh=mesh,
        out_type=jax.ShapeDtypeStruct((num_indices, embed_dim), table.dtype),
        scratch_types=[
            pltpu.VMEM((per_worker,), jnp.int32),
            pltpu.VMEM((2, W, embed_dim), jnp.float32),
            pltpu.SemaphoreType.DMA,
            pltpu.SemaphoreType.DMA,
            pltpu.SemaphoreType.DMA,
            pltpu.SemaphoreType.DMA,
        ],
        compiler_params=pltpu.CompilerParams(use_tc_tiling_on_sc=False),
    )
    def sc_gather(table_hbm, idx_hbm, out_hbm, idx_v, rows_v, g0, g1, o0, o1):
        wid = lax.axis_index("s") * NC + lax.axis_index("c")
        base = wid * per_worker
        gsem = (g0, g1)
        osem = (o0, o1)
        pltpu.sync_copy(idx_hbm.at[pl.ds(base, per_worker)], idx_v)

        def gather_cp(buf, t):
            return pltpu.make_async_copy(
                table_hbm.at[idx_v.at[pl.ds(t * W, W)]],
                rows_v.at[buf],
                gsem[buf],
            )

        def out_cp(buf, t):
            return pltpu.make_async_copy(
                rows_v.at[buf],
                out_hbm.at[pl.ds(base + t * W, W)],
                osem[buf],
            )

        gather_cp(0, 0).start()
        gather_cp(1, 1).start()

        @pl.loop(0, n_batches // 2 - 1)
        def _(h):
            t0 = 2 * h
            gather_cp(0, t0).wait()
            out_cp(0, t0).start()
            gather_cp(1, t0 + 1).wait()
            out_cp(1, t0 + 1).start()
            out_cp(0, t0).wait()
            gather_cp(0, t0 + 2).start()
            out_cp(1, t0 + 1).wait()
            gather_cp(1, t0 + 3).start()

        tl = n_batches - 2
        gather_cp(0, tl).wait()
        out_cp(0, tl).start()
        gather_cp(1, tl + 1).wait()
        out_cp(1, tl + 1).start()
        out_cp(0, tl).wait()
        out_cp(1, tl + 1).wait()

    out = sc_gather(table, idx_flat)
    return out.reshape(batch, seq, embed_dim)
